# Initial kernel scaffold; baseline (speedup 1.0000x reference)
import jax
import jax.numpy as jnp
from jax.experimental import pallas as pl


def kernel(tokens, timestamps, channel_embed, pos_embed, month_table):
    raise NotImplementedError("write your pallas kernel here")


# speedup vs baseline: 2.5616x; 2.5616x over previous
import jax
import jax.numpy as jnp
from jax.experimental import pallas as pl
from jax.experimental.pallas import tpu as pltpu


def _body(months_ref, tokens_ref, ch_ref, pos_ref, month_ref, out_ref):
    ch = ch_ref[...]
    pe = jnp.broadcast_to(pos_ref[0], (3, 192))
    me = jnp.broadcast_to(month_ref[0], (3, 192))
    zero = jnp.zeros((3, 192), jnp.float32)
    addend = jnp.concatenate([ch, pe, me, zero], axis=-1)
    out_ref[...] = tokens_ref[...] + addend[None, None, None, None, :, :]


def kernel(tokens, timestamps, channel_embed, pos_embed, month_table):
    b, h, w, t, bs, d = tokens.shape
    n = d // 4
    months = timestamps[:, :, 1].astype(jnp.int32)
    pos3 = pos_embed.reshape(pos_embed.shape[0], 1, n)
    month3 = month_table.reshape(month_table.shape[0], 1, n)

    grid_spec = pltpu.PrefetchScalarGridSpec(
        num_scalar_prefetch=1,
        grid=(b, t),
        in_specs=[
            pl.BlockSpec((1, h, w, 1, bs, d), lambda i, j, m: (i, 0, 0, j, 0, 0)),
            pl.BlockSpec((bs, n), lambda i, j, m: (0, 0)),
            pl.BlockSpec((1, 1, n), lambda i, j, m: (j, 0, 0)),
            pl.BlockSpec((1, 1, n), lambda i, j, m: (m[i, j], 0, 0)),
        ],
        out_specs=pl.BlockSpec((1, h, w, 1, bs, d), lambda i, j, m: (i, 0, 0, j, 0, 0)),
    )
    return pl.pallas_call(
        _body,
        grid_spec=grid_spec,
        out_shape=jax.ShapeDtypeStruct(tokens.shape, tokens.dtype),
    )(months, tokens, channel_embed, pos3, month3)
